# baseline (speedup 1.0000x reference)
import jax
import jax.numpy as jnp
from jax.experimental import pallas as pl

_B = 65536
_N_THR = 8
_HID = 32
_N_OUT = 9
_BLK = 512


def _dot(a, b):
    return jax.lax.dot_general(a, b, (((1,), (0,)), ((), ())),
                               preferred_element_type=jnp.float32)


def _avg_stat(x, avg_bf):
    hi = x.astype(jnp.bfloat16)
    lo = (x - hi.astype(jnp.float32)).astype(jnp.bfloat16)
    return _dot(hi, avg_bf) + _dot(lo, avg_bf)


def _ln_grouped(a, avg, g_t, be_t):
    mu = _avg_stat(a, avg)
    asq = _avg_stat(a * a, avg)
    var = asq - mu * mu
    return (a - mu) * jax.lax.rsqrt(var + 1e-5) * g_t + be_t


def _softplus(x):
    return jnp.maximum(x, 0.0) + jnp.log1p(jnp.exp(-jnp.abs(x)))


def _bd(w, copies):
    return jnp.kron(jnp.eye(copies, dtype=w.dtype), w)


def _t2(v):
    return jnp.tile(v.reshape(1, -1), (1, 2))


def _split_rows(w, sizes):
    out = []
    off = 0
    for s in sizes:
        out.append(w[off:off + s])
        off += s
    return tuple(out)


def _prep(params):
    p = {}
    enc = params['enc']
    p['enc'] = {
        'W1': _bd(enc['W1'], 2), 'b1': _t2(enc['b1']),
        'beta1': enc['beta1'].reshape(1, 1),
        'rbeta1': (1.0 / enc['beta1']).reshape(1, 1),
        'g1': _t2(enc['g1']), 'be1': _t2(enc['be1']),
        'W2': _bd(enc['W2'], 2), 'b2': _t2(enc['b2']),
    }
    layers = []
    for lp in params['layers']:
        t = lp['thr']
        wt, we, wh = _split_rows(t['W1'], (8, 7, _HID))
        we8 = jnp.concatenate([we, jnp.zeros((1, _HID), jnp.float32)], 0)
        thr = {
            'W1t': _bd(wt, 16),
            'W1e': _bd(we8, 16),
            'W1h': _bd(jnp.tile(wh, (1, _N_THR)), 2),
            'b1': jnp.tile(t['b1'].reshape(1, -1), (1, 16)),
            'beta1': t['beta1'].reshape(1, 1),
            'rbeta1': (1.0 / t['beta1']).reshape(1, 1),
            'g1': jnp.tile(t['g1'].reshape(1, -1), (1, 16)),
            'be1': jnp.tile(t['be1'].reshape(1, -1), (1, 16)),
            'W2s': _bd(jnp.tile(t['W2'], (_N_THR, 1)), 2),
            'b2x8': _t2(t['b2'] * float(_N_THR)),
        }
        h, b = lp['hyd'], lp['buo']
        hx, hea, hh = _split_rows(h['W1'], (4, 8, _HID))
        bx, bea, bh = _split_rows(b['W1'], (6, 4, _HID))
        z32 = jnp.zeros((_HID, _HID), jnp.float32)
        cat = jnp.concatenate
        one32 = jnp.full((_HID,), 1.0, jnp.float32)
        hb = {
            'W1hx': _bd(cat([hx, jnp.zeros((4, _HID), jnp.float32)], 1), 2),
            'W1hea': _bd(cat([hea, jnp.zeros((8, _HID), jnp.float32)], 1), 2),
            'W1bx': _bd(cat([jnp.zeros((6, _HID), jnp.float32), bx], 1), 2),
            'W1bea': _bd(cat([jnp.zeros((4, _HID), jnp.float32), bea], 1), 2),
            'W1h': _bd(cat([hh, bh], 1), 2),
            'b1': _t2(cat([h['b1'], b['b1']])),
            'beta': _t2(cat([one32 * h['beta1'], one32 * b['beta1']])),
            'rbeta': _t2(cat([one32 / h['beta1'], one32 / b['beta1']])),
            'g1': _t2(cat([h['g1'], b['g1']])),
            'be1': _t2(cat([h['be1'], b['be1']])),
            'W2': _bd(cat([cat([h['W2'], z32], 1),
                           cat([z32, b['W2']], 1)], 0), 2),
            'b2': _t2(cat([h['b2'], b['b2']])),
        }
        u = lp['upd']
        uh, ua, um1, um2 = _split_rows(u['W1'], (_HID,) * 4)
        upd = {
            'W1h': _bd(uh, 2), 'W1a': _bd(ua, 2),
            'W1m': _bd(jnp.concatenate([um1, um2], 0), 2),
            'b1': _t2(u['b1']),
            'beta1': u['beta1'].reshape(1, 1),
            'rbeta1': (1.0 / u['beta1']).reshape(1, 1),
            'g1': _t2(u['g1']), 'be1': _t2(u['be1']),
            'W2': _bd(u['W2'], 2), 'b2': _t2(u['b2']),
        }
        layers.append({'thr': thr, 'hb': hb, 'upd': upd})
    r = params['readout']
    p['layers'] = layers
    j32 = jnp.full((_HID, _HID), 1.0 / _HID, jnp.bfloat16)
    p['avg64'] = _bd(j32, 2)
    p['avg128'] = _bd(j32, 4)
    p['avg512'] = _bd(j32, 16)
    p['readout'] = {
        'W1': _bd(r['W1'], 2), 'b1': _t2(r['b1']),
        'beta1': r['beta1'].reshape(1, 1), 'rbeta1': (1.0 / r['beta1']).reshape(1, 1),
        'g1': _t2(r['g1']), 'be1': _t2(r['be1']),
        'W2': _bd(r['W2'], 2), 'b2': _t2(r['b2']),
        'beta2': r['beta2'].reshape(1, 1), 'rbeta2': (1.0 / r['beta2']).reshape(1, 1),
        'g2': _t2(r['g2']), 'be2': _t2(r['be2']),
        'W3': _bd(r['W3'], 2), 'b3': _t2(r['b3']),
    }
    return p


def _pignn_body(z_ref, thr_ref, hyd_ref, buo_ref, eat_ref, eah_ref, eab_ref,
                prep_refs, out_ref):
    P = jax.tree.map(lambda r: r[...], prep_refs)
    m = _BLK

    z = z_ref[...]
    pe = P['enc']
    a = _softplus(pe['beta1'] * (_dot(z, pe['W1']) + pe['b1'])) * pe['rbeta1']
    hull = _dot(_ln_grouped(a, P['avg64'], pe['g1'], pe['be1']),
                pe['W2']) + pe['b2']

    thr = thr_ref[...]
    eat = eat_ref[...]
    hyd = hyd_ref[...]
    eah = eah_ref[...]
    buo = buo_ref[...]
    eab = eab_ref[...]

    for li in range(2):
        L = P['layers'][li]
        pt = L['thr']
        pre = (_dot(thr, pt['W1t']) + _dot(eat, pt['W1e'])
               + _dot(hull, pt['W1h']) + pt['b1'])
        a = _softplus(pt['beta1'] * pre) * pt['rbeta1']
        y = _ln_grouped(a, P['avg512'], pt['g1'], pt['be1'])
        agg = _dot(y, pt['W2s']) + pt['b2x8']
        ph = L['hb']
        pre = (_dot(hyd, ph['W1hx']) + _dot(eah, ph['W1hea'])
               + _dot(buo, ph['W1bx']) + _dot(eab, ph['W1bea'])
               + _dot(hull, ph['W1h']) + ph['b1'])
        a = _softplus(ph['beta'] * pre) * ph['rbeta']
        y = _ln_grouped(a, P['avg128'], ph['g1'], ph['be1'])
        msg_hb = _dot(y, ph['W2']) + ph['b2']
        pu = L['upd']
        pre = (_dot(hull, pu['W1h']) + _dot(agg, pu['W1a'])
               + _dot(msg_hb, pu['W1m']) + pu['b1'])
        a = _softplus(pu['beta1'] * pre) * pu['rbeta1']
        hull = _dot(_ln_grouped(a, P['avg64'], pu['g1'], pu['be1']),
                    pu['W2']) + pu['b2']

    pr = P['readout']
    a = _softplus(pr['beta1'] * (_dot(hull, pr['W1']) + pr['b1'])) * pr['rbeta1']
    h = _dot(_ln_grouped(a, P['avg64'], pr['g1'], pr['be1']), pr['W2']) + pr['b2']
    a = _softplus(pr['beta2'] * h) * pr['rbeta2']
    delta = _dot(_ln_grouped(a, P['avg64'], pr['g2'], pr['be2']),
                 pr['W3']) + pr['b3']

    state = jnp.concatenate([z[:, 0:_N_OUT], z[:, 14:14 + _N_OUT]], axis=1)
    xh = delta + state
    col = jax.lax.broadcasted_iota(jnp.int32, (m, 2 * _N_OUT), 1)
    out = xh
    for half in range(2):
        d = half * _N_OUT
        cos = xh[:, d + 3:d + 4]
        sin = xh[:, d + 4:d + 5]
        d0 = delta[:, d:d + 1]
        d1 = delta[:, d + 1:d + 2]
        s0 = state[:, d:d + 1]
        s1 = state[:, d + 1:d + 2]
        xw = cos * d0 - sin * d1 + s0
        yw = sin * d0 + cos * d1 + s1
        out = jnp.where(col == d, xw, jnp.where(col == d + 1, yw, out))
    out_ref[...] = out


@jax.jit
def kernel(Z, thruster_x, hydro_x, buoy_x, ea_t, ea_h, ea_b, params):
    prep = _prep(params)
    half = _B // 2
    z2 = Z.reshape(half, 28)
    thr2 = thruster_x.reshape(half, 128)
    ea8 = jnp.pad(ea_t, ((0, 0), (0, 1)))
    eat2 = ea8.reshape(half, 128)
    hyd2 = hydro_x.reshape(half, 8)
    eah2 = ea_h.reshape(half, 16)
    buo2 = buoy_x.reshape(half, 12)
    eab2 = ea_b.reshape(half, 8)

    m = _BLK
    grid = half // m

    def row_spec(rows, cols):
        return pl.BlockSpec((rows, cols), lambda i: (i, 0))

    prep_specs = jax.tree.map(
        lambda a: pl.BlockSpec(a.shape, lambda i: (0,) * a.ndim), prep)

    out2 = pl.pallas_call(
        _pignn_body,
        grid=(grid,),
        in_specs=[
            row_spec(m, 28),
            row_spec(m, 128),
            row_spec(m, 8),
            row_spec(m, 12),
            row_spec(m, 128),
            row_spec(m, 16),
            row_spec(m, 8),
            prep_specs,
        ],
        out_specs=row_spec(m, 2 * _N_OUT),
        out_shape=jax.ShapeDtypeStruct((half, 2 * _N_OUT), jnp.float32),
    )(z2, thr2, hyd2, buo2, eat2, eah2, eab2, prep)
    return out2.reshape(_B, _N_OUT)


# speedup vs baseline: 1.1181x; 1.1181x over previous
import jax
import jax.numpy as jnp
from jax.experimental import pallas as pl

_B = 65536
_N_THR = 8
_HID = 32
_N_OUT = 9
_BLK = 512


def _dot(a, b):
    return jax.lax.dot_general(a, b, (((1,), (0,)), ((), ())),
                               preferred_element_type=jnp.float32)


def _avg_stat(x, avg_bf):
    hi = x.astype(jnp.bfloat16)
    lo = (x - hi.astype(jnp.float32)).astype(jnp.bfloat16)
    return _dot(hi, avg_bf) + _dot(lo, avg_bf)


def _ln_grouped(a, avg, g_t, be_t):
    mu = _avg_stat(a, avg)
    asq = _avg_stat(a * a, avg)
    var = asq - mu * mu
    return (a - mu) * jax.lax.rsqrt(var + 1e-5) * g_t + be_t


def _softplus(x):
    return jnp.maximum(x, 0.0) + jnp.log1p(jnp.exp(-jnp.abs(x)))


def _bd(w, copies):
    return jnp.kron(jnp.eye(copies, dtype=w.dtype), w)


def _t2(v):
    return jnp.tile(v.reshape(1, -1), (1, 2))


def _split_rows(w, sizes):
    out = []
    off = 0
    for s in sizes:
        out.append(w[off:off + s])
        off += s
    return tuple(out)


def _prep(params):
    p = {}
    enc = params['enc']
    p['enc'] = {
        'W1': _bd(enc['W1'], 2), 'b1': _t2(enc['b1']),
        'beta1': enc['beta1'].reshape(1, 1),
        'rbeta1': (1.0 / enc['beta1']).reshape(1, 1),
        'g1': _t2(enc['g1']), 'be1': _t2(enc['be1']),
        'W2': _bd(enc['W2'], 2), 'b2': _t2(enc['b2']),
    }
    layers = []
    for lp in params['layers']:
        t = lp['thr']
        wt, we, wh = _split_rows(t['W1'], (8, 7, _HID))
        thr = {
            'W1t': _bd(wt, 16),
            'W1e': _bd(we, 16),
            'W1h': _bd(jnp.tile(wh, (1, _N_THR)), 2),
            'b1': jnp.tile(t['b1'].reshape(1, -1), (1, 16)),
            'beta1': t['beta1'].reshape(1, 1),
            'rbeta1': (1.0 / t['beta1']).reshape(1, 1),
            'g1': jnp.tile(t['g1'].reshape(1, -1), (1, 16)),
            'be1': jnp.tile(t['be1'].reshape(1, -1), (1, 16)),
            'W2s': _bd(jnp.tile(t['W2'], (_N_THR, 1)), 2),
            'b2x8': _t2(t['b2'] * float(_N_THR)),
        }
        h, b = lp['hyd'], lp['buo']
        hx, hea, hh = _split_rows(h['W1'], (4, 8, _HID))
        bx, bea, bh = _split_rows(b['W1'], (6, 4, _HID))
        z32 = jnp.zeros((_HID, _HID), jnp.float32)
        cat = jnp.concatenate
        one32 = jnp.full((_HID,), 1.0, jnp.float32)
        hb = {
            'W1hx': _bd(cat([hx, jnp.zeros((4, _HID), jnp.float32)], 1), 2),
            'W1hea': _bd(cat([hea, jnp.zeros((8, _HID), jnp.float32)], 1), 2),
            'W1bx': _bd(cat([jnp.zeros((6, _HID), jnp.float32), bx], 1), 2),
            'W1bea': _bd(cat([jnp.zeros((4, _HID), jnp.float32), bea], 1), 2),
            'W1h': _bd(cat([hh, bh], 1), 2),
            'b1': _t2(cat([h['b1'], b['b1']])),
            'beta': _t2(cat([one32 * h['beta1'], one32 * b['beta1']])),
            'rbeta': _t2(cat([one32 / h['beta1'], one32 / b['beta1']])),
            'g1': _t2(cat([h['g1'], b['g1']])),
            'be1': _t2(cat([h['be1'], b['be1']])),
            'W2': _bd(cat([cat([h['W2'], z32], 1),
                           cat([z32, b['W2']], 1)], 0), 2),
            'b2': _t2(cat([h['b2'], b['b2']])),
        }
        u = lp['upd']
        uh, ua, um1, um2 = _split_rows(u['W1'], (_HID,) * 4)
        upd = {
            'W1h': _bd(uh, 2), 'W1a': _bd(ua, 2),
            'W1m': _bd(jnp.concatenate([um1, um2], 0), 2),
            'b1': _t2(u['b1']),
            'beta1': u['beta1'].reshape(1, 1),
            'rbeta1': (1.0 / u['beta1']).reshape(1, 1),
            'g1': _t2(u['g1']), 'be1': _t2(u['be1']),
            'W2': _bd(u['W2'], 2), 'b2': _t2(u['b2']),
        }
        layers.append({'thr': thr, 'hb': hb, 'upd': upd})
    r = params['readout']
    p['layers'] = layers
    j32 = jnp.full((_HID, _HID), 1.0 / _HID, jnp.bfloat16)
    p['avg64'] = _bd(j32, 2)
    p['avg128'] = _bd(j32, 4)
    p['avg512'] = _bd(j32, 16)
    p['readout'] = {
        'W1': _bd(r['W1'], 2), 'b1': _t2(r['b1']),
        'beta1': r['beta1'].reshape(1, 1), 'rbeta1': (1.0 / r['beta1']).reshape(1, 1),
        'g1': _t2(r['g1']), 'be1': _t2(r['be1']),
        'W2': _bd(r['W2'], 2), 'b2': _t2(r['b2']),
        'beta2': r['beta2'].reshape(1, 1), 'rbeta2': (1.0 / r['beta2']).reshape(1, 1),
        'g2': _t2(r['g2']), 'be2': _t2(r['be2']),
        'W3': _bd(r['W3'], 2), 'b3': _t2(r['b3']),
    }
    return p


def _pignn_body(z_ref, thr_ref, hyd_ref, buo_ref, eat_ref, eah_ref, eab_ref,
                prep_refs, out_ref):
    P = jax.tree.map(lambda r: r[...], prep_refs)
    m = _BLK

    z = z_ref[...]
    pe = P['enc']
    a = _softplus(pe['beta1'] * (_dot(z, pe['W1']) + pe['b1'])) * pe['rbeta1']
    hull = _dot(_ln_grouped(a, P['avg64'], pe['g1'], pe['be1']),
                pe['W2']) + pe['b2']

    thr = thr_ref[...]
    eat = eat_ref[...]
    hyd = hyd_ref[...]
    eah = eah_ref[...]
    buo = buo_ref[...]
    eab = eab_ref[...]

    for li in range(2):
        L = P['layers'][li]
        pt = L['thr']
        pre = (_dot(thr, pt['W1t']) + _dot(eat, pt['W1e'])
               + _dot(hull, pt['W1h']) + pt['b1'])
        a = _softplus(pt['beta1'] * pre) * pt['rbeta1']
        y = _ln_grouped(a, P['avg512'], pt['g1'], pt['be1'])
        agg = _dot(y, pt['W2s']) + pt['b2x8']
        ph = L['hb']
        pre = (_dot(hyd, ph['W1hx']) + _dot(eah, ph['W1hea'])
               + _dot(buo, ph['W1bx']) + _dot(eab, ph['W1bea'])
               + _dot(hull, ph['W1h']) + ph['b1'])
        a = _softplus(ph['beta'] * pre) * ph['rbeta']
        y = _ln_grouped(a, P['avg128'], ph['g1'], ph['be1'])
        msg_hb = _dot(y, ph['W2']) + ph['b2']
        pu = L['upd']
        pre = (_dot(hull, pu['W1h']) + _dot(agg, pu['W1a'])
               + _dot(msg_hb, pu['W1m']) + pu['b1'])
        a = _softplus(pu['beta1'] * pre) * pu['rbeta1']
        hull = _dot(_ln_grouped(a, P['avg64'], pu['g1'], pu['be1']),
                    pu['W2']) + pu['b2']

    pr = P['readout']
    a = _softplus(pr['beta1'] * (_dot(hull, pr['W1']) + pr['b1'])) * pr['rbeta1']
    h = _dot(_ln_grouped(a, P['avg64'], pr['g1'], pr['be1']), pr['W2']) + pr['b2']
    a = _softplus(pr['beta2'] * h) * pr['rbeta2']
    delta = _dot(_ln_grouped(a, P['avg64'], pr['g2'], pr['be2']),
                 pr['W3']) + pr['b3']

    state = jnp.concatenate([z[:, 0:_N_OUT], z[:, 14:14 + _N_OUT]], axis=1)
    xh = delta + state
    col = jax.lax.broadcasted_iota(jnp.int32, (m, 2 * _N_OUT), 1)
    out = xh
    for half in range(2):
        d = half * _N_OUT
        cos = xh[:, d + 3:d + 4]
        sin = xh[:, d + 4:d + 5]
        d0 = delta[:, d:d + 1]
        d1 = delta[:, d + 1:d + 2]
        s0 = state[:, d:d + 1]
        s1 = state[:, d + 1:d + 2]
        xw = cos * d0 - sin * d1 + s0
        yw = sin * d0 + cos * d1 + s1
        out = jnp.where(col == d, xw, jnp.where(col == d + 1, yw, out))
    out_ref[...] = out


@jax.jit
def kernel(Z, thruster_x, hydro_x, buoy_x, ea_t, ea_h, ea_b, params):
    prep = _prep(params)
    half = _B // 2
    z2 = Z.reshape(half, 28)
    thr2 = thruster_x.reshape(half, 128)
    eat2 = ea_t.reshape(half, 112)
    hyd2 = hydro_x.reshape(half, 8)
    eah2 = ea_h.reshape(half, 16)
    buo2 = buoy_x.reshape(half, 12)
    eab2 = ea_b.reshape(half, 8)

    m = _BLK
    grid = half // m

    def row_spec(rows, cols):
        return pl.BlockSpec((rows, cols), lambda i: (i, 0))

    prep_specs = jax.tree.map(
        lambda a: pl.BlockSpec(a.shape, lambda i: (0,) * a.ndim), prep)

    out2 = pl.pallas_call(
        _pignn_body,
        grid=(grid,),
        in_specs=[
            row_spec(m, 28),
            row_spec(m, 128),
            row_spec(m, 8),
            row_spec(m, 12),
            row_spec(m, 112),
            row_spec(m, 16),
            row_spec(m, 8),
            prep_specs,
        ],
        out_specs=row_spec(m, 2 * _N_OUT),
        out_shape=jax.ShapeDtypeStruct((half, 2 * _N_OUT), jnp.float32),
    )(z2, thr2, hyd2, buo2, eat2, eah2, eab2, prep)
    return out2.reshape(_B, _N_OUT)
